# baseline (speedup 1.0000x reference)
import jax
import jax.numpy as jnp
from jax.experimental import pallas as pl
from jax.experimental.pallas import tpu as pltpu

_ROWS = 64
_COLS = 8192
_RBLK = 8
_N = _ROWS // _RBLK
_INF = float("inf")


def _tc_body(xl_any, xu_any, outl_ref, outu_ref, xlv, xuv, *sems):
    rows = lambda i: pl.ds(i * _RBLK, _RBLK)
    cps = []
    for i in range(_N):
        cpl = pltpu.make_async_copy(xl_any.at[rows(i)], xlv.at[rows(i)],
                                    sems[2 * i])
        cpu = pltpu.make_async_copy(xu_any.at[rows(i)], xuv.at[rows(i)],
                                    sems[2 * i + 1])
        cpl.start()
        cpu.start()
        cps.append((cpl, cpu))

    for i in range(_N):
        cpl, cpu = cps[i]
        cpl.wait()
        cpu.wait()
        xl = xlv[rows(i), :]
        xu = xuv[rows(i), :]
        ka = jnp.float32(0.7) * xl + jnp.float32(0.3) * xu
        minka = jnp.min(ka, axis=1, keepdims=True)
        kb = jnp.float32(0.3) * xl + jnp.float32(0.7) * xu
        kbm = jnp.where(ka == minka, kb, _INF)
        minkb = jnp.min(kbm, axis=1, keepdims=True)
        sel = kbm == minkb
        outl_ref[rows(i), :] = jnp.min(jnp.where(sel, xl, _INF), axis=1,
                                       keepdims=True)
        outu_ref[rows(i), :] = jnp.min(jnp.where(sel, xu, _INF), axis=1,
                                       keepdims=True)


@jax.jit
def kernel(xl, xu):
    return pl.pallas_call(
        _tc_body,
        in_specs=[
            pl.BlockSpec(memory_space=pl.ANY),
            pl.BlockSpec(memory_space=pl.ANY),
        ],
        out_shape=(
            jax.ShapeDtypeStruct((_ROWS, 1), jnp.float32),
            jax.ShapeDtypeStruct((_ROWS, 1), jnp.float32),
        ),
        scratch_shapes=[
            pltpu.VMEM((_ROWS, _COLS), jnp.float32),
            pltpu.VMEM((_ROWS, _COLS), jnp.float32),
        ] + [pltpu.SemaphoreType.DMA] * (2 * _N),
    )(xl, xu)


# speedup vs baseline: 1.1770x; 1.1770x over previous
import jax
import jax.numpy as jnp
from jax.experimental import pallas as pl
from jax.experimental.pallas import tpu as pltpu

_ROWS = 64
_COLS = 8192
_RBLK = 32
_N = _ROWS // _RBLK
_INF = float("inf")


def _tc_body(xl_any, xu_any, outl_ref, outu_ref, xlv, xuv, *sems):
    rows = lambda i: pl.ds(i * _RBLK, _RBLK)
    cps = []
    for i in range(_N):
        cpl = pltpu.make_async_copy(xl_any.at[rows(i)], xlv.at[rows(i)],
                                    sems[2 * i])
        cpu = pltpu.make_async_copy(xu_any.at[rows(i)], xuv.at[rows(i)],
                                    sems[2 * i + 1])
        cpl.start()
        cpu.start()
        cps.append((cpl, cpu))

    for i in range(_N):
        cpl, cpu = cps[i]
        cpl.wait()
        cpu.wait()
        xl = xlv[rows(i), :]
        xu = xuv[rows(i), :]
        ka = jnp.float32(0.7) * xl + jnp.float32(0.3) * xu
        minka = jnp.min(ka, axis=1, keepdims=True)
        kb = jnp.float32(0.3) * xl + jnp.float32(0.7) * xu
        kbm = jnp.where(ka == minka, kb, _INF)
        minkb = jnp.min(kbm, axis=1, keepdims=True)
        sel = kbm == minkb
        outl_ref[rows(i), :] = jnp.min(jnp.where(sel, xl, _INF), axis=1,
                                       keepdims=True)
        outu_ref[rows(i), :] = jnp.min(jnp.where(sel, xu, _INF), axis=1,
                                       keepdims=True)


@jax.jit
def kernel(xl, xu):
    return pl.pallas_call(
        _tc_body,
        in_specs=[
            pl.BlockSpec(memory_space=pl.ANY),
            pl.BlockSpec(memory_space=pl.ANY),
        ],
        out_shape=(
            jax.ShapeDtypeStruct((_ROWS, 1), jnp.float32),
            jax.ShapeDtypeStruct((_ROWS, 1), jnp.float32),
        ),
        scratch_shapes=[
            pltpu.VMEM((_ROWS, _COLS), jnp.float32),
            pltpu.VMEM((_ROWS, _COLS), jnp.float32),
        ] + [pltpu.SemaphoreType.DMA] * (2 * _N),
    )(xl, xu)
